# Initial kernel scaffold; baseline (speedup 1.0000x reference)
import jax
import jax.numpy as jnp
from jax.experimental import pallas as pl


def kernel(stops, x, x_dist, x_features, x_week, x_mask, emb_week, emb_stop, fc1_W, fc1_b, Wl, bl, Wr):
    raise NotImplementedError("write your pallas kernel here")


# speedup vs baseline: 6.0618x; 6.0618x over previous
import dataclasses
import functools

import jax
import jax.numpy as jnp
from jax import lax
from jax.experimental import pallas as pl
from jax.experimental.pallas import tpu as pltpu
from jax.experimental.pallas import tpu_sc as plsc

_B, _NN, _LB, _TRAJ = 64, 10000, 8, 64


def _sc_kernel(stops, xflat, table, pack):
    mesh = plsc.VectorSubcoreMesh(core_axis_name="c", subcore_axis_name="s")
    cp = pltpu.CompilerParams(use_tc_tiling_on_sc=False)
    if "needs_layout_passes" in pltpu.CompilerParams.__dataclass_fields__:
        cp = dataclasses.replace(cp, needs_layout_passes=False)

    out_type = (
        jax.ShapeDtypeStruct((_B, _NN), jnp.float32),
        jax.ShapeDtypeStruct((_B, _NN), jnp.float32),
        jax.ShapeDtypeStruct((_B, 16), jnp.float32),
    )

    @functools.partial(
        pl.kernel,
        mesh=mesh,
        out_type=out_type,
        scratch_types=[
            pltpu.VMEM((_TRAJ,), jnp.int32),
            pltpu.VMEM((_TRAJ, 16), jnp.float32),
            pltpu.VMEM((4, 128), jnp.int32),
            pltpu.VMEM((512,), jnp.float32),
            pltpu.VMEM((_NN,), jnp.float32),
            pltpu.VMEM((_NN,), jnp.float32),
            pltpu.VMEM((16,), jnp.float32),
            pltpu.VMEM((16,), jnp.float32),
            pltpu.SemaphoreType.DMA,
        ],
        compiler_params=cp,
    )
    def sck(stops_hbm, xflat_hbm, table_hbm, pack_hbm,
            aggv_hbm, cnt_hbm, xe_hbm,
            sb, rows, idxb, xv, aggl, cntl, packb, xebuf, sem):
        wid = lax.axis_index("s") * 2 + lax.axis_index("c")
        pltpu.sync_copy(pack_hbm, packb)
        pv = packb[...]
        iota = lax.iota(jnp.int32, 16)

        def pack_scalar(lane):
            return jnp.sum(jnp.where(iota == lane, pv, jnp.float32(0.0)))

        wvec = [pack_scalar(l) for l in range(_LB)]
        fc1b = pack_scalar(8)
        wl48 = pack_scalar(9)
        wl49 = pack_scalar(10)
        ones16 = jnp.ones((16,), jnp.float32)
        lane12 = jnp.full((16,), 12, jnp.int32)

        for r in range(2):
            b = wid * 2 + r
            pltpu.sync_copy(stops_hbm.at[b], sb)
            pltpu.async_copy(table_hbm.at[sb], rows, sem).wait()
            acc = jnp.zeros((16,), jnp.float32)
            for t in range(_TRAJ):
                acc = acc + rows[t]
            xebuf[...] = acc
            pltpu.sync_copy(xebuf, xe_hbm.at[b])

            base = b * (_LB * _NN)
            for c in range(32):
                l = c // 4
                toff = (c % 4) * 16
                sbc = sb[pl.ds(toff, 16)]
                idxb[c // 8, pl.ds((c % 8) * 16, 16)] = sbc + (base + l * _NN)
            for j in range(4):
                pltpu.async_copy(xflat_hbm.at[idxb.at[j]],
                                 xv.at[pl.ds(j * 128, 128)], sem).wait()

            @pl.loop(0, _NN, step=16)
            def _(i):
                z = jnp.zeros((16,), jnp.float32)
                aggl[pl.ds(i, 16)] = z
                cntl[pl.ds(i, 16)] = z

            for c in range(4):
                toff = c * 16
                osrc = jnp.full((16,), fc1b)
                for l in range(_LB):
                    osrc = osrc + xv[pl.ds(l * _TRAJ + toff, 16)] * wvec[l]
                xdv = plsc.load_gather(rows, [iota + toff, lane12])
                v = osrc * wl48 + xdv * wl49
                valid = (iota + toff) < (_TRAJ - 1)
                dstv = plsc.load_gather(
                    sb, [jnp.minimum(iota + (toff + 1), _TRAJ - 1)])
                for j in range(16):
                    m = valid & (iota == j)
                    plsc.addupdate_scatter(aggl, [dstv], v, mask=m)
                    plsc.addupdate_scatter(cntl, [dstv], ones16, mask=m)

            pltpu.sync_copy(aggl, aggv_hbm.at[b])
            pltpu.sync_copy(cntl, cnt_hbm.at[b])

    return sck(stops, xflat, table, pack)


def _tc_kernel(x, aggv, cnt, xe, x_week, x_mask, emb_week, x_features,
               x_dist2, fc1_W, fc1_b, Wl, bl, Wr):
    def body(xw_ref, fb_ref, bl_ref, x_ref, aggv_ref, cnt_ref, xe_ref,
             mask_ref, ew_ref, xf_ref, xd_ref, fw_ref, wl_ref, wr_ref, o_ref):
        b = pl.program_id(0)
        wk = xw_ref[b]
        ew = ew_ref[pl.ds(wk, 1), :][0]
        xfr = xf_ref[pl.ds(b, 1), :][0]
        xer = xe_ref[pl.ds(b, 1), :][0]
        wl = wl_ref[0, :]
        wr = wr_ref[0, :]
        cL = (jnp.sum(ew * wl[0:34]) + jnp.sum(xfr * wl[34:36])
              + jnp.sum(xer[0:12] * wl[36:48]))
        cR = (jnp.sum(ew * wr[0:34]) + jnp.sum(xfr * wr[34:36])
              + jnp.sum(xer[0:12] * wr[36:48]))
        xb = x_ref[0]
        outr = jnp.sum(xb * fw_ref[0, :][:, None], axis=0) + fb_ref[0]
        cntr = cnt_ref[0, 0]
        mc = (aggv_ref[0, 0] / jnp.maximum(cntr, 1.0)
              + cL * (cntr >= 0.5).astype(jnp.float32))
        g = (mc + bl_ref[0] + cR + outr * wr[48:49]
             + xd_ref[0] * wr[49:50])
        gm = jnp.max(g)
        logp = (g - gm) - jnp.log(jnp.sum(jnp.exp(g - gm)))
        o_ref[0, 0, :] = jnp.where(mask_ref[0, 0] != 0, -1e8, logp)

    return pl.pallas_call(
        body,
        grid=(_B,),
        in_specs=[
            pl.BlockSpec(memory_space=pltpu.SMEM),
            pl.BlockSpec(memory_space=pltpu.SMEM),
            pl.BlockSpec(memory_space=pltpu.SMEM),
            pl.BlockSpec((1, _LB, _NN), lambda b: (b, 0, 0)),
            pl.BlockSpec((1, 1, _NN), lambda b: (b, 0, 0)),
            pl.BlockSpec((1, 1, _NN), lambda b: (b, 0, 0)),
            pl.BlockSpec((_B, 16), lambda b: (0, 0)),
            pl.BlockSpec((1, 1, _NN), lambda b: (b, 0, 0)),
            pl.BlockSpec((7, 34), lambda b: (0, 0)),
            pl.BlockSpec((_B, 2), lambda b: (0, 0)),
            pl.BlockSpec((1, _NN), lambda b: (0, 0)),
            pl.BlockSpec((1, _LB), lambda b: (0, 0)),
            pl.BlockSpec((1, 50), lambda b: (0, 0)),
            pl.BlockSpec((1, 50), lambda b: (0, 0)),
        ],
        out_specs=pl.BlockSpec((1, 1, _NN), lambda b: (b, 0, 0)),
        out_shape=jax.ShapeDtypeStruct((_B, 1, _NN), jnp.float32),
    )(x_week, fc1_b, bl, x, aggv.reshape(_B, 1, _NN),
      cnt.reshape(_B, 1, _NN), xe, x_mask.reshape(_B, 1, _NN), emb_week,
      x_features, x_dist2, fc1_W, Wl, Wr).reshape(_B, _NN)


def kernel(stops, x, x_dist, x_features, x_week, x_mask, emb_week, emb_stop,
           fc1_W, fc1_b, Wl, bl, Wr):
    stops32 = stops.astype(jnp.int32)
    x_week32 = x_week.astype(jnp.int32)
    xflat = x.reshape(-1)
    table = jnp.concatenate(
        [emb_stop, x_dist[:, None],
         jnp.zeros((_NN, 3), jnp.float32)], axis=1)
    pack = jnp.concatenate(
        [fc1_W[0], fc1_b, Wl[0, 48:50], jnp.zeros((5,), jnp.float32)])
    aggv, cnt, xe = _sc_kernel(stops32, xflat, table, pack)
    return _tc_kernel(x, aggv, cnt, xe, x_week32, x_mask, emb_week,
                      x_features, x_dist.reshape(1, _NN), fc1_W, fc1_b, Wl,
                      bl, Wr)


# speedup vs baseline: 10.5958x; 1.7480x over previous
import dataclasses
import functools

import jax
import jax.numpy as jnp
from jax import lax
from jax.experimental import pallas as pl
from jax.experimental.pallas import tpu as pltpu
from jax.experimental.pallas import tpu_sc as plsc

_B, _NN, _LB, _TRAJ = 64, 10000, 8, 64


def _tc_a(x, fc1_W, fc1_b):
    def body(fb_ref, x_ref, fw_ref, o_ref):
        w = fw_ref[0, :]
        o_ref[...] = jnp.sum(x_ref[...] * w[None, :, None], axis=1) + fb_ref[0]

    return pl.pallas_call(
        body,
        grid=(8,),
        in_specs=[
            pl.BlockSpec(memory_space=pltpu.SMEM),
            pl.BlockSpec((8, _LB, _NN), lambda i: (i, 0, 0)),
            pl.BlockSpec((1, _LB), lambda i: (0, 0)),
        ],
        out_specs=pl.BlockSpec((8, _NN), lambda i: (i, 0)),
        out_shape=jax.ShapeDtypeStruct((_B, _NN), jnp.float32),
    )(fc1_b, x, fc1_W)


def _sc_kernel(stops, outT, table, pack, x_week, x_feat_flat, emb_week_flat):
    mesh = plsc.VectorSubcoreMesh(core_axis_name="c", subcore_axis_name="s")
    cp = pltpu.CompilerParams(use_tc_tiling_on_sc=False)
    if "needs_layout_passes" in pltpu.CompilerParams.__dataclass_fields__:
        cp = dataclasses.replace(cp, needs_layout_passes=False)

    out_type = (
        jax.ShapeDtypeStruct((_B, _NN), jnp.float32),
        jax.ShapeDtypeStruct((_B, _NN), jnp.float32),
        jax.ShapeDtypeStruct((_B, 16), jnp.float32),
    )

    @functools.partial(
        pl.kernel,
        mesh=mesh,
        out_type=out_type,
        scratch_types=[
            pltpu.VMEM((_TRAJ,), jnp.int32),
            pltpu.VMEM((_TRAJ, 16), jnp.float32),
            pltpu.VMEM((_TRAJ, _B), jnp.float32),
            pltpu.VMEM((_NN,), jnp.float32),
            pltpu.VMEM((_NN,), jnp.float32),
            pltpu.VMEM((144,), jnp.float32),
            pltpu.VMEM((64,), jnp.int32),
            pltpu.VMEM((128,), jnp.float32),
            pltpu.VMEM((240,), jnp.float32),
            pltpu.VMEM((16,), jnp.float32),
            pltpu.SemaphoreType.DMA,
        ],
        compiler_params=cp,
    )
    def sck(stops_hbm, outT_hbm, table_hbm, pack_hbm, xw_hbm, xf_hbm, ew_hbm,
            aggv_hbm, cnt_hbm, clcr_hbm,
            sb, rows, orows, aggl, cntl, packb, xwbuf, xfbuf, ewbuf, clbuf,
            sem):
        wid = lax.axis_index("s") * 2 + lax.axis_index("c")
        pltpu.sync_copy(pack_hbm, packb)
        pltpu.sync_copy(xw_hbm, xwbuf)
        pltpu.sync_copy(xf_hbm, xfbuf)
        pltpu.sync_copy(ew_hbm, ewbuf)
        iota = lax.iota(jnp.int32, 16)
        p0 = packb[pl.ds(0, 16)]
        wl48 = jnp.sum(jnp.where(iota == 0, p0, 0.0))
        wl49 = jnp.sum(jnp.where(iota == 1, p0, 0.0))
        ones16 = jnp.ones((16,), jnp.float32)
        lane12 = jnp.full((16,), 12, jnp.int32)
        masks = [(iota == j) for j in range(16)]
        wlA = packb[pl.ds(16, 16)]
        wlB = packb[pl.ds(32, 16)]
        wlC = packb[pl.ds(48, 16)]
        wlE = packb[pl.ds(64, 16)]
        wrA = packb[pl.ds(80, 16)]
        wrB = packb[pl.ds(96, 16)]
        wrC = packb[pl.ds(112, 16)]
        wrE = packb[pl.ds(128, 16)]

        for r in range(2):
            b = wid * 2 + r
            bv = jnp.full((16,), 0, jnp.int32) + b
            pltpu.sync_copy(stops_hbm.at[b], sb)
            pltpu.async_copy(table_hbm.at[sb], rows, sem).wait()
            pltpu.async_copy(outT_hbm.at[sb], orows, sem).wait()
            acc = jnp.zeros((16,), jnp.float32)
            for t in range(_TRAJ):
                acc = acc + rows[t]

            wv = plsc.load_gather(xwbuf, [bv]) * 34
            ew0 = plsc.load_gather(ewbuf, [wv + iota])
            ew1 = plsc.load_gather(ewbuf, [wv + (iota + 16)])
            ew2 = plsc.load_gather(
                ewbuf, [jnp.minimum(wv + (iota + 32), 237)])
            xfg = plsc.load_gather(
                xfbuf, [jnp.clip(iota + (2 * b - 2), 0, 127)])
            chunk2 = jnp.where(iota < 2, ew2,
                               jnp.where(iota < 4, xfg, 0.0))
            cl = jnp.sum(ew0 * wlA + ew1 * wlB + chunk2 * wlC + acc * wlE)
            cr = jnp.sum(ew0 * wrA + ew1 * wrB + chunk2 * wrC + acc * wrE)
            clbuf[...] = jnp.where(iota == 0, cl,
                                   jnp.where(iota == 1, cr, 0.0))
            pltpu.sync_copy(clbuf, clcr_hbm.at[b])

            @pl.loop(0, _NN, step=400)
            def _(i):
                z = jnp.zeros((16,), jnp.float32)
                for u in range(25):
                    aggl[pl.ds(i + u * 16, 16)] = z
                    cntl[pl.ds(i + u * 16, 16)] = z

            for c in range(4):
                toff = c * 16
                osrc = plsc.load_gather(orows, [iota + toff, bv])
                xdv = plsc.load_gather(rows, [iota + toff, lane12])
                v = osrc * wl48 + xdv * wl49
                valid = (iota + toff) < (_TRAJ - 1)
                dstv = plsc.load_gather(
                    sb, [jnp.minimum(iota + (toff + 1), _TRAJ - 1)])
                for j in range(16):
                    m = valid & masks[j]
                    plsc.addupdate_scatter(aggl, [dstv], v, mask=m)
                    plsc.addupdate_scatter(cntl, [dstv], ones16, mask=m)

            pltpu.sync_copy(aggl, aggv_hbm.at[b])
            pltpu.sync_copy(cntl, cnt_hbm.at[b])

    return sck(stops, outT, table, pack, x_week, x_feat_flat, emb_week_flat)


def _tc_b(out2d, aggv, cnt, clcr, x_mask, x_dist2, Wr, bl):
    def body(bl_ref, out_ref, aggv_ref, cnt_ref, clcr_ref, mask_ref, xd_ref,
             wr_ref, o_ref):
        wr = wr_ref[0, :]
        clb = clcr_ref[:, 0:1]
        crb = clcr_ref[:, 1:2]
        cntb = cnt_ref[...]
        g = (aggv_ref[...] / jnp.maximum(cntb, 1.0)
             + clb * (cntb >= 0.5).astype(jnp.float32)
             + bl_ref[0] + crb + out_ref[...] * wr[48:49]
             + xd_ref[0, :][None, :] * wr[49:50])
        gm = jnp.max(g, axis=1, keepdims=True)
        e = jnp.exp(g - gm)
        logp = (g - gm) - jnp.log(jnp.sum(e, axis=1, keepdims=True))
        o_ref[...] = jnp.where(mask_ref[...] != 0, -1e8, logp)

    return pl.pallas_call(
        body,
        grid=(8,),
        in_specs=[
            pl.BlockSpec(memory_space=pltpu.SMEM),
            pl.BlockSpec((8, _NN), lambda i: (i, 0)),
            pl.BlockSpec((8, _NN), lambda i: (i, 0)),
            pl.BlockSpec((8, _NN), lambda i: (i, 0)),
            pl.BlockSpec((8, 16), lambda i: (i, 0)),
            pl.BlockSpec((8, _NN), lambda i: (i, 0)),
            pl.BlockSpec((1, _NN), lambda i: (0, 0)),
            pl.BlockSpec((1, 50), lambda i: (0, 0)),
        ],
        out_specs=pl.BlockSpec((8, _NN), lambda i: (i, 0)),
        out_shape=jax.ShapeDtypeStruct((_B, _NN), jnp.float32),
    )(bl, out2d, aggv, cnt, clcr, x_mask, x_dist2, Wr)


def kernel(stops, x, x_dist, x_features, x_week, x_mask, emb_week, emb_stop,
           fc1_W, fc1_b, Wl, bl, Wr):
    f32 = jnp.float32
    stops32 = stops.astype(jnp.int32)
    x_week32 = x_week.astype(jnp.int32)
    table = jnp.concatenate(
        [emb_stop, x_dist[:, None], jnp.zeros((_NN, 3), f32)], axis=1)
    z12 = jnp.zeros((12,), f32)
    z4 = jnp.zeros((4,), f32)
    pack = jnp.concatenate([
        Wl[0, 48:50], jnp.zeros((14,), f32),
        Wl[0, 0:16], Wl[0, 16:32], Wl[0, 32:36], z12, Wl[0, 36:48], z4,
        Wr[0, 0:16], Wr[0, 16:32], Wr[0, 32:36], z12, Wr[0, 36:48], z4,
    ])
    xf_flat = x_features.reshape(-1).astype(f32)
    ew_flat = jnp.concatenate([emb_week.reshape(-1), jnp.zeros((2,), f32)])

    out2d = _tc_a(x, fc1_W, fc1_b)
    outT = out2d.T
    aggv, cnt, clcr = _sc_kernel(stops32, outT, table, pack, x_week32,
                                 xf_flat, ew_flat)
    return _tc_b(out2d, aggv, cnt, clcr, x_mask, x_dist.reshape(1, _NN),
                 Wr, bl)


# speedup vs baseline: 11.4746x; 1.0829x over previous
import dataclasses
import functools

import jax
import jax.numpy as jnp
from jax import lax
from jax.experimental import pallas as pl
from jax.experimental.pallas import tpu as pltpu
from jax.experimental.pallas import tpu_sc as plsc

_B, _NN, _LB, _TRAJ = 64, 10000, 8, 64


def _tc_a(x, fc1_W, fc1_b):
    def body(fb_ref, x_ref, fw_ref, o_ref):
        w = fw_ref[0, :]
        o_ref[...] = jnp.sum(x_ref[...] * w[None, :, None], axis=1) + fb_ref[0]

    return pl.pallas_call(
        body,
        grid=(4,),
        in_specs=[
            pl.BlockSpec(memory_space=pltpu.SMEM),
            pl.BlockSpec((16, _LB, _NN), lambda i: (i, 0, 0)),
            pl.BlockSpec((1, _LB), lambda i: (0, 0)),
        ],
        out_specs=pl.BlockSpec((16, _NN), lambda i: (i, 0)),
        out_shape=jax.ShapeDtypeStruct((_B, _NN), jnp.float32),
    )(fc1_b, x, fc1_W)


def _sc_kernel(stops, out_flat, table, pack, x_week, x_feat_flat,
               emb_week_flat):
    mesh = plsc.VectorSubcoreMesh(core_axis_name="c", subcore_axis_name="s")
    cp = pltpu.CompilerParams(use_tc_tiling_on_sc=False)
    if "needs_layout_passes" in pltpu.CompilerParams.__dataclass_fields__:
        cp = dataclasses.replace(cp, needs_layout_passes=False)

    out_type = (
        jax.ShapeDtypeStruct((_B, _NN), jnp.float32),
        jax.ShapeDtypeStruct((_B, _NN), jnp.float32),
        jax.ShapeDtypeStruct((_B, 16), jnp.float32),
    )

    @functools.partial(
        pl.kernel,
        mesh=mesh,
        out_type=out_type,
        scratch_types=[
            pltpu.VMEM((_TRAJ,), jnp.int32),
            pltpu.VMEM((_TRAJ,), jnp.int32),
            pltpu.VMEM((_TRAJ, 16), jnp.float32),
            pltpu.VMEM((_TRAJ,), jnp.float32),
            pltpu.VMEM((_NN,), jnp.float32),
            pltpu.VMEM((_NN,), jnp.float32),
            pltpu.VMEM((_NN,), jnp.float32),
            pltpu.VMEM((_NN,), jnp.float32),
            pltpu.VMEM((144,), jnp.float32),
            pltpu.VMEM((64,), jnp.int32),
            pltpu.VMEM((128,), jnp.float32),
            pltpu.VMEM((240,), jnp.float32),
            pltpu.VMEM((16,), jnp.float32),
            pltpu.SemaphoreType.DMA,
            pltpu.SemaphoreType.DMA,
            pltpu.SemaphoreType.DMA,
        ],
        compiler_params=cp,
    )
    def sck(stops_hbm, oflat_hbm, table_hbm, pack_hbm, xw_hbm, xf_hbm,
            ew_hbm, aggv_hbm, cnt_hbm, clcr_hbm,
            sb, oidx, rows, osrc_v, aggl0, cntl0, aggl1, cntl1, packb,
            xwbuf, xfbuf, ewbuf, clbuf, sem_rows, sem_osrc, sem_out):
        wid = lax.axis_index("s") * 2 + lax.axis_index("c")
        pltpu.sync_copy(pack_hbm, packb)
        pltpu.sync_copy(xw_hbm, xwbuf)
        pltpu.sync_copy(xf_hbm, xfbuf)
        pltpu.sync_copy(ew_hbm, ewbuf)
        iota = lax.iota(jnp.int32, 16)
        p0 = packb[pl.ds(0, 16)]
        wl48 = jnp.sum(jnp.where(iota == 0, p0, 0.0))
        wl49 = jnp.sum(jnp.where(iota == 1, p0, 0.0))
        ones16 = jnp.ones((16,), jnp.float32)
        lane12 = jnp.full((16,), 12, jnp.int32)
        masks = [(iota == j) for j in range(16)]
        wlA = packb[pl.ds(16, 16)]
        wlB = packb[pl.ds(32, 16)]
        wlC = packb[pl.ds(48, 16)]
        wlE = packb[pl.ds(64, 16)]
        wrA = packb[pl.ds(80, 16)]
        wrB = packb[pl.ds(96, 16)]
        wrC = packb[pl.ds(112, 16)]
        wrE = packb[pl.ds(128, 16)]

        out_copies = []
        for r, (aggl, cntl) in enumerate(((aggl0, cntl0), (aggl1, cntl1))):
            b = wid * 2 + r
            bv = jnp.full((16,), 0, jnp.int32) + b
            pltpu.sync_copy(stops_hbm.at[b], sb)
            rows_cp = pltpu.async_copy(table_hbm.at[sb], rows, sem_rows)
            boff = b * _NN
            for u in range(4):
                oidx[pl.ds(u * 16, 16)] = sb[pl.ds(u * 16, 16)] + boff
            osrc_cp = pltpu.async_copy(oflat_hbm.at[oidx], osrc_v, sem_osrc)
            rows_cp.wait()
            acc = jnp.zeros((16,), jnp.float32)
            for t in range(_TRAJ):
                acc = acc + rows[t]

            wv = plsc.load_gather(xwbuf, [bv]) * 34
            ew0 = plsc.load_gather(ewbuf, [wv + iota])
            ew1 = plsc.load_gather(ewbuf, [wv + (iota + 16)])
            ew2 = plsc.load_gather(
                ewbuf, [jnp.minimum(wv + (iota + 32), 237)])
            xfg = plsc.load_gather(
                xfbuf, [jnp.clip(iota + (2 * b - 2), 0, 127)])
            chunk2 = jnp.where(iota < 2, ew2,
                               jnp.where(iota < 4, xfg, 0.0))
            cl = jnp.sum(ew0 * wlA + ew1 * wlB + chunk2 * wlC + acc * wlE)
            cr = jnp.sum(ew0 * wrA + ew1 * wrB + chunk2 * wrC + acc * wrE)
            clbuf[...] = jnp.where(iota == 0, cl,
                                   jnp.where(iota == 1, cr, 0.0))
            pltpu.sync_copy(clbuf, clcr_hbm.at[b])

            @pl.loop(0, _NN, step=400)
            def _(i):
                z = jnp.zeros((16,), jnp.float32)
                for u in range(25):
                    aggl[pl.ds(i + u * 16, 16)] = z
                    cntl[pl.ds(i + u * 16, 16)] = z

            osrc_cp.wait()
            for c in range(4):
                toff = c * 16
                osrc = osrc_v[pl.ds(toff, 16)]
                xdv = plsc.load_gather(rows, [iota + toff, lane12])
                v = osrc * wl48 + xdv * wl49
                valid = (iota + toff) < (_TRAJ - 1)
                dstv = plsc.load_gather(
                    sb, [jnp.minimum(iota + (toff + 1), _TRAJ - 1)])
                for j in range(16):
                    m = valid & masks[j]
                    plsc.addupdate_scatter(aggl, [dstv], v, mask=m)
                    plsc.addupdate_scatter(cntl, [dstv], ones16, mask=m)

            out_copies.append(
                pltpu.async_copy(aggl, aggv_hbm.at[b], sem_out))
            out_copies.append(
                pltpu.async_copy(cntl, cnt_hbm.at[b], sem_out))

        for c in out_copies:
            c.wait()

    return sck(stops, out_flat, table, pack, x_week, x_feat_flat,
               emb_week_flat)


def _tc_b(out2d, aggv, cnt, clcr, x_mask, x_dist2, Wr, bl):
    def body(bl_ref, out_ref, aggv_ref, cnt_ref, clcr_ref, mask_ref, xd_ref,
             wr_ref, o_ref):
        wr = wr_ref[0, :]
        clb = clcr_ref[:, 0:1]
        crb = clcr_ref[:, 1:2]
        cntb = cnt_ref[...]
        g = (aggv_ref[...] / jnp.maximum(cntb, 1.0)
             + clb * (cntb >= 0.5).astype(jnp.float32)
             + bl_ref[0] + crb + out_ref[...] * wr[48:49]
             + xd_ref[0, :][None, :] * wr[49:50])
        gm = jnp.max(g, axis=1, keepdims=True)
        e = jnp.exp(g - gm)
        logp = (g - gm) - jnp.log(jnp.sum(e, axis=1, keepdims=True))
        o_ref[...] = jnp.where(mask_ref[...] != 0, -1e8, logp)

    return pl.pallas_call(
        body,
        grid=(8,),
        in_specs=[
            pl.BlockSpec(memory_space=pltpu.SMEM),
            pl.BlockSpec((8, _NN), lambda i: (i, 0)),
            pl.BlockSpec((8, _NN), lambda i: (i, 0)),
            pl.BlockSpec((8, _NN), lambda i: (i, 0)),
            pl.BlockSpec((8, 16), lambda i: (i, 0)),
            pl.BlockSpec((8, _NN), lambda i: (i, 0)),
            pl.BlockSpec((1, _NN), lambda i: (0, 0)),
            pl.BlockSpec((1, 50), lambda i: (0, 0)),
        ],
        out_specs=pl.BlockSpec((8, _NN), lambda i: (i, 0)),
        out_shape=jax.ShapeDtypeStruct((_B, _NN), jnp.float32),
    )(bl, out2d, aggv, cnt, clcr, x_mask, x_dist2, Wr)


def kernel(stops, x, x_dist, x_features, x_week, x_mask, emb_week, emb_stop,
           fc1_W, fc1_b, Wl, bl, Wr):
    f32 = jnp.float32
    stops32 = stops.astype(jnp.int32)
    x_week32 = x_week.astype(jnp.int32)
    table = jnp.concatenate(
        [emb_stop, x_dist[:, None], jnp.zeros((_NN, 3), f32)], axis=1)
    z12 = jnp.zeros((12,), f32)
    z4 = jnp.zeros((4,), f32)
    pack = jnp.concatenate([
        Wl[0, 48:50], jnp.zeros((14,), f32),
        Wl[0, 0:16], Wl[0, 16:32], Wl[0, 32:36], z12, Wl[0, 36:48], z4,
        Wr[0, 0:16], Wr[0, 16:32], Wr[0, 32:36], z12, Wr[0, 36:48], z4,
    ])
    xf_flat = x_features.reshape(-1).astype(f32)
    ew_flat = jnp.concatenate([emb_week.reshape(-1), jnp.zeros((2,), f32)])

    out2d = _tc_a(x, fc1_W, fc1_b)
    aggv, cnt, clcr = _sc_kernel(stops32, out2d.reshape(-1), table, pack,
                                 x_week32, xf_flat, ew_flat)
    return _tc_b(out2d, aggv, cnt, clcr, x_mask, x_dist.reshape(1, _NN),
                 Wr, bl)
